# baseline (speedup 1.0000x reference)
import functools

import jax
import jax.numpy as jnp
from jax import lax
from jax.experimental import pallas as pl
from jax.experimental.pallas import tpu as pltpu
from jax.experimental.pallas import tpu_sc as plsc

N_USERS = 1000000
N_ITEMS = 1000000
FACTORS = 16
BATCH = 16384

NC = 2
NS = 16
L = 16
NW = NC * NS
BPW = BATCH // NW

_mesh = plsc.VectorSubcoreMesh(core_axis_name="c", subcore_axis_name="s")


@functools.partial(
    pl.kernel,
    out_type=jax.ShapeDtypeStruct((BATCH,), jnp.float32),
    mesh=_mesh,
    scratch_types=[
        pltpu.VMEM((BPW,), jnp.int32),
        pltpu.VMEM((BPW,), jnp.int32),
        pltpu.VMEM((BPW, FACTORS), jnp.float32),
        pltpu.VMEM((BPW, FACTORS), jnp.float32),
        pltpu.VMEM((BPW,), jnp.float32),
        pltpu.SemaphoreType.DMA,
    ],
    compiler_params=pltpu.CompilerParams(
        needs_layout_passes=False, use_tc_tiling_on_sc=False),
)
def _sc_dot(u_hbm, i_hbm, ut_hbm, it_hbm, out_hbm,
            uidx_v, iidx_v, urows_v, irows_v, out_v, sem):
    wid = lax.axis_index("s") * NC + lax.axis_index("c")
    base = wid * BPW

    pltpu.sync_copy(u_hbm.at[pl.ds(base, BPW)], uidx_v)
    pltpu.sync_copy(i_hbm.at[pl.ds(base, BPW)], iidx_v)

    cu = pltpu.async_copy(ut_hbm.at[uidx_v], urows_v, sem)
    ci = pltpu.async_copy(it_hbm.at[iidx_v], irows_v, sem)
    cu.wait()
    ci.wait()

    def group(j, carry):
        rows = j * L + lax.iota(jnp.int32, L)
        acc = jnp.zeros((L,), jnp.float32)
        for f in range(FACTORS):
            col = jnp.full((L,), f, jnp.int32)
            uv = plsc.load_gather(urows_v, [rows, col])
            iv = plsc.load_gather(irows_v, [rows, col])
            acc = acc + uv * iv
        out_v[pl.ds(j * L, L)] = acc
        return carry

    lax.fori_loop(0, BPW // L, group, 0)

    pltpu.sync_copy(out_v, out_hbm.at[pl.ds(base, BPW)])


def kernel(u, i, user_table, item_table):
    out = _sc_dot(u, i, user_table, item_table)
    return out.reshape(BATCH, 1, 1)


# speedup vs baseline: 1.4822x; 1.4822x over previous
import functools

import jax
import jax.numpy as jnp
from jax import lax
from jax.experimental import pallas as pl
from jax.experimental.pallas import tpu as pltpu
from jax.experimental.pallas import tpu_sc as plsc

N_USERS = 1000000
N_ITEMS = 1000000
FACTORS = 16
BATCH = 16384

NC = 2
NS = 16
L = 16
NW = NC * NS
BPW = BATCH // NW
CHUNK = 256

_mesh = plsc.VectorSubcoreMesh(core_axis_name="c", subcore_axis_name="s")


@functools.partial(
    pl.kernel,
    out_type=jax.ShapeDtypeStruct((BATCH,), jnp.float32),
    mesh=_mesh,
    scratch_types=[
        pltpu.VMEM((BPW,), jnp.int32),
        pltpu.VMEM((BPW,), jnp.int32),
        pltpu.VMEM((CHUNK, FACTORS), jnp.float32),
        pltpu.VMEM((CHUNK, FACTORS), jnp.float32),
        pltpu.VMEM((BPW,), jnp.float32),
        pltpu.SemaphoreType.DMA,
    ],
    compiler_params=pltpu.CompilerParams(needs_layout_passes=False),
)
def _sc_dot(u_hbm, i_hbm, ut_hbm, it_hbm, out_hbm,
            uidx_v, iidx_v, urows_v, irows_v, out_v, sem):
    wid = lax.axis_index("s") * NC + lax.axis_index("c")
    base = wid * BPW

    pltpu.sync_copy(u_hbm.at[pl.ds(base, BPW)], uidx_v)
    pltpu.sync_copy(i_hbm.at[pl.ds(base, BPW)], iidx_v)

    for c in range(BPW // CHUNK):
        coff = c * CHUNK

        def fire(g, carry):
            k0 = g * L
            uvec = uidx_v[pl.ds(coff + k0, L)]
            ivec = iidx_v[pl.ds(coff + k0, L)]
            for t in range(L):
                pltpu.async_copy(ut_hbm.at[uvec[t]], urows_v.at[k0 + t], sem)
                pltpu.async_copy(it_hbm.at[ivec[t]], irows_v.at[k0 + t], sem)
            return carry

        lax.fori_loop(0, CHUNK // L, fire, 0)

        pltpu.make_async_copy(ut_hbm.at[pl.ds(0, CHUNK)], urows_v, sem).wait()
        pltpu.make_async_copy(ut_hbm.at[pl.ds(0, CHUNK)], irows_v, sem).wait()

        def group(j, carry):
            rows = j * L + lax.iota(jnp.int32, L)
            acc = jnp.zeros((L,), jnp.float32)
            for f in range(FACTORS):
                col = jnp.full((L,), f, jnp.int32)
                uv = plsc.load_gather(urows_v, [rows, col])
                iv = plsc.load_gather(irows_v, [rows, col])
                acc = acc + uv * iv
            out_v[pl.ds(coff + j * L, L)] = acc
            return carry

        lax.fori_loop(0, CHUNK // L, group, 0)

    pltpu.sync_copy(out_v, out_hbm.at[pl.ds(base, BPW)])


def kernel(u, i, user_table, item_table):
    out = _sc_dot(u, i, user_table, item_table)
    return out.reshape(BATCH, 1, 1)


# speedup vs baseline: 1.4859x; 1.0025x over previous
import functools

import jax
import jax.numpy as jnp
from jax import lax
from jax.experimental import pallas as pl
from jax.experimental.pallas import tpu as pltpu
from jax.experimental.pallas import tpu_sc as plsc

N_USERS = 1000000
N_ITEMS = 1000000
FACTORS = 16
BATCH = 16384

NC = 2
NS = 16
L = 16
NW = NC * NS
BPW = BATCH // NW
CHUNK = 256
NSEM = 8

_mesh = plsc.VectorSubcoreMesh(core_axis_name="c", subcore_axis_name="s")


@functools.partial(
    pl.kernel,
    out_type=jax.ShapeDtypeStruct((BATCH,), jnp.float32),
    mesh=_mesh,
    scratch_types=[
        pltpu.VMEM((BPW,), jnp.int32),
        pltpu.VMEM((BPW,), jnp.int32),
        pltpu.VMEM((CHUNK, FACTORS), jnp.float32),
        pltpu.VMEM((CHUNK, FACTORS), jnp.float32),
        pltpu.VMEM((BPW,), jnp.float32),
        [pltpu.SemaphoreType.DMA] * NSEM,
    ],
    compiler_params=pltpu.CompilerParams(needs_layout_passes=False),
)
def _sc_dot(u_hbm, i_hbm, ut_hbm, it_hbm, out_hbm,
            uidx_v, iidx_v, urows_v, irows_v, out_v, sems):
    wid = lax.axis_index("s") * NC + lax.axis_index("c")
    base = wid * BPW

    pltpu.sync_copy(u_hbm.at[pl.ds(base, BPW)], uidx_v)
    pltpu.sync_copy(i_hbm.at[pl.ds(base, BPW)], iidx_v)

    for c in range(BPW // CHUNK):
        coff = c * CHUNK

        def fire(g, carry):
            k0 = g * L
            uvec = uidx_v[pl.ds(coff + k0, L)]
            ivec = iidx_v[pl.ds(coff + k0, L)]
            for t in range(L):
                pltpu.async_copy(
                    ut_hbm.at[uvec[t]], urows_v.at[k0 + t], sems[t % NSEM])
                pltpu.async_copy(
                    it_hbm.at[ivec[t]], irows_v.at[k0 + t], sems[t % NSEM])
            return carry

        lax.fori_loop(0, CHUNK // L, fire, 0)

        per_sem_rows = 2 * CHUNK // NSEM
        for s in range(NSEM):
            pltpu.make_async_copy(
                ut_hbm.at[pl.ds(0, per_sem_rows)],
                urows_v.at[pl.ds(0, per_sem_rows)], sems[s]).wait()

        def group(j, carry):
            rows = j * L + lax.iota(jnp.int32, L)
            acc = jnp.zeros((L,), jnp.float32)
            for f in range(FACTORS):
                col = jnp.full((L,), f, jnp.int32)
                uv = plsc.load_gather(urows_v, [rows, col])
                iv = plsc.load_gather(irows_v, [rows, col])
                acc = acc + uv * iv
            out_v[pl.ds(coff + j * L, L)] = acc
            return carry

        lax.fori_loop(0, CHUNK // L, group, 0)

    pltpu.sync_copy(out_v, out_hbm.at[pl.ds(base, BPW)])


def kernel(u, i, user_table, item_table):
    out = _sc_dot(u, i, user_table, item_table)
    return out.reshape(BATCH, 1, 1)
